# Initial kernel scaffold; baseline (speedup 1.0000x reference)
import jax
import jax.numpy as jnp
from jax.experimental import pallas as pl


def kernel(x, edge_index, W1, b1, Wl, Wr, bS, Wa, ba, Wb, bb, Wc, bc):
    raise NotImplementedError("write your pallas kernel here")


# speedup vs baseline: 8.4724x; 8.4724x over previous
import functools

import jax
import jax.numpy as jnp
from jax import lax
from jax.experimental import pallas as pl
from jax.experimental.pallas import tpu as pltpu
from jax.experimental.pallas import tpu_sc as plsc

N = 10000
E = 320000
D = 128
NC = 2
NS = 16
NW = NC * NS
GROUP = 128
G = 2560
EPAD = G * GROUP
GPT = G // NW
NACC = 10240
RPT = NACC // NS
ZROWS = 128
LANES = 16

_mesh = plsc.VectorSubcoreMesh(
    core_axis_name="c", subcore_axis_name="s", num_cores=NC, num_subcores=NS
)


DEGW = 128


@functools.partial(
    pl.kernel,
    out_type=jax.ShapeDtypeStruct((NC, NACC, DEGW), jnp.float32),
    mesh=_mesh,
    scratch_types=[
        pltpu.VMEM((GPT, GROUP), jnp.int32),
        pltpu.VMEM((ZROWS, DEGW), jnp.float32),
        pltpu.VMEM_SHARED((NACC, DEGW), jnp.float32),
    ],
)
def _deg_kernel(dstg_hbm, out_hbm, dst_v, buf_v, acc_sh):
    c = lax.axis_index("c")
    s = lax.axis_index("s")
    wid = s * NC + c
    pltpu.sync_copy(dstg_hbm.at[pl.ds(wid * GPT, GPT)], dst_v)

    zvec = jnp.zeros((LANES,), jnp.float32)
    ovec = jnp.ones((LANES,), jnp.float32)

    def fill_zero(i, carry):
        for j in range(DEGW // LANES):
            buf_v[i, pl.ds(j * LANES, LANES)] = zvec
        return carry

    lax.fori_loop(0, ZROWS, fill_zero, 0)
    for k in range(RPT // ZROWS):
        pltpu.sync_copy(buf_v, acc_sh.at[pl.ds(s * RPT + k * ZROWS, ZROWS)])
    plsc.subcore_barrier()

    def fill_ones(i, carry):
        for j in range(DEGW // LANES):
            buf_v[i, pl.ds(j * LANES, LANES)] = ovec
        return carry

    lax.fori_loop(0, ZROWS, fill_ones, 0)

    def edge_body(i, carry):
        pltpu.sync_copy(buf_v, acc_sh.at[dst_v.at[i]], add=True)
        return carry

    lax.fori_loop(0, GPT, edge_body, 0)
    plsc.subcore_barrier()
    for k in range(RPT // ZROWS):
        r0 = s * RPT + k * ZROWS
        pltpu.sync_copy(acc_sh.at[pl.ds(r0, ZROWS)],
                        out_hbm.at[c, pl.ds(r0, ZROWS)])


@functools.partial(
    pl.kernel,
    out_type=jax.ShapeDtypeStruct((NC, NACC, D), jnp.float32),
    mesh=_mesh,
    scratch_types=[
        pltpu.VMEM((GPT, GROUP), jnp.int32),
        pltpu.VMEM((GPT, GROUP), jnp.int32),
        pltpu.VMEM((GROUP, D), jnp.float32),
        pltpu.VMEM_SHARED((NACC, D), jnp.float32),
        pltpu.SemaphoreType.DMA,
    ],
)
def _scatter_kernel(table_hbm, srcg_hbm, dstg_hbm, out_hbm,
                    src_v, dst_v, rows_v, acc_sh, sem):
    c = lax.axis_index("c")
    s = lax.axis_index("s")
    wid = s * NC + c

    zvec = jnp.zeros((LANES,), jnp.float32)

    def zb(i, carry):
        for j in range(D // LANES):
            rows_v[i, pl.ds(j * LANES, LANES)] = zvec
        return carry

    lax.fori_loop(0, ZROWS, zb, 0)
    for k in range(RPT // ZROWS):
        pltpu.sync_copy(rows_v, acc_sh.at[pl.ds(s * RPT + k * ZROWS, ZROWS)])
    plsc.subcore_barrier()

    pltpu.sync_copy(srcg_hbm.at[pl.ds(wid * GPT, GPT)], src_v)
    pltpu.sync_copy(dstg_hbm.at[pl.ds(wid * GPT, GPT)], dst_v)

    def group_body(i, carry):
        pltpu.async_copy(table_hbm.at[src_v.at[i]], rows_v, sem).wait()
        pltpu.sync_copy(rows_v, acc_sh.at[dst_v.at[i]], add=True)
        return carry

    lax.fori_loop(0, GPT, group_body, 0)
    plsc.subcore_barrier()

    for k in range(RPT // ZROWS):
        r0 = s * RPT + k * ZROWS
        pltpu.sync_copy(acc_sh.at[pl.ds(r0, ZROWS)],
                        out_hbm.at[c, pl.ds(r0, ZROWS)])


RB = 1000


def _mm1_body(x_ref, w_ref, dv_ref, o_ref):
    o_ref[...] = (
        jnp.dot(x_ref[...], w_ref[...], preferred_element_type=jnp.float32)
        * dv_ref[...]
    )


def _mm1(x, W1, dinvM):
    return pl.pallas_call(
        _mm1_body,
        grid=(N // RB,),
        in_specs=[
            pl.BlockSpec((RB, D), lambda i: (i, 0)),
            pl.BlockSpec((D, D), lambda i: (0, 0)),
            pl.BlockSpec((RB, D), lambda i: (i, 0)),
        ],
        out_specs=pl.BlockSpec((RB, D), lambda i: (i, 0)),
        out_shape=jax.ShapeDtypeStruct((N, D), jnp.float32),
    )(x, W1, dinvM)


def _ew1_body(p0_ref, p1_ref, hs_ref, dv_ref, b1_ref, o_ref):
    agg = dv_ref[...] * (p0_ref[...] + p1_ref[...] + hs_ref[...])
    o_ref[...] = jnp.maximum(agg + b1_ref[...], 0.0)


def _ew1(p0, p1, hs, dinvM, b1):
    return pl.pallas_call(
        _ew1_body,
        grid=(N // RB,),
        in_specs=[
            pl.BlockSpec((RB, D), lambda i: (i, 0)),
            pl.BlockSpec((RB, D), lambda i: (i, 0)),
            pl.BlockSpec((RB, D), lambda i: (i, 0)),
            pl.BlockSpec((RB, D), lambda i: (i, 0)),
            pl.BlockSpec((1, D), lambda i: (0, 0)),
        ],
        out_specs=pl.BlockSpec((RB, D), lambda i: (i, 0)),
        out_shape=jax.ShapeDtypeStruct((N, D), jnp.float32),
    )(p0, p1, hs, dinvM, b1)


D_MID = 64
D_OUT = 16


def _mlp_body(p0_ref, p1_ref, h1_ref, Wl_ref, Wr_ref, bS_ref,
              Wa_ref, ba_ref, Wb_ref, bb_ref, Wc_ref, bc_ref, o_ref):
    nb = p0_ref[...] + p1_ref[...]
    h2 = jnp.maximum(
        jnp.dot(nb, Wl_ref[...], preferred_element_type=jnp.float32)
        + jnp.dot(h1_ref[...], Wr_ref[...], preferred_element_type=jnp.float32)
        + bS_ref[...],
        0.0,
    )
    z = jnp.maximum(
        jnp.dot(h2, Wa_ref[...], preferred_element_type=jnp.float32)
        + ba_ref[...],
        0.0,
    )
    z = jnp.maximum(
        jnp.dot(z, Wb_ref[...], preferred_element_type=jnp.float32)
        + bb_ref[...],
        0.0,
    )
    o_ref[...] = (
        jnp.dot(z, Wc_ref[...], preferred_element_type=jnp.float32)
        + bc_ref[...]
    )


def _mlp(p0, p1, h1, Wl, Wr, bS, Wa, ba, Wb, bb, Wc, bc):
    row = lambda i: (i, 0)
    fix = lambda i: (0, 0)
    return pl.pallas_call(
        _mlp_body,
        grid=(N // RB,),
        in_specs=[
            pl.BlockSpec((RB, D), row),
            pl.BlockSpec((RB, D), row),
            pl.BlockSpec((RB, D), row),
            pl.BlockSpec((D, D), fix),
            pl.BlockSpec((D, D), fix),
            pl.BlockSpec((1, D), fix),
            pl.BlockSpec((D, D_MID), fix),
            pl.BlockSpec((1, D_MID), fix),
            pl.BlockSpec((D_MID, D), fix),
            pl.BlockSpec((1, D), fix),
            pl.BlockSpec((D, D_OUT), fix),
            pl.BlockSpec((1, D_OUT), fix),
        ],
        out_specs=pl.BlockSpec((RB, D_OUT), row),
        out_shape=jax.ShapeDtypeStruct((N, D_OUT), jnp.float32),
    )(p0, p1, h1, Wl, Wr, bS, Wa, ba, Wb, bb, Wc, bc)


def kernel(x, edge_index, W1, b1, Wl, Wr, bS, Wa, ba, Wb, bb, Wc, bc):
    src = edge_index[0]
    dst = edge_index[1]
    pad = EPAD - E
    srcg = jnp.concatenate([src, jnp.zeros((pad,), jnp.int32)]).reshape(G, GROUP)
    dstg = jnp.concatenate([dst, jnp.full((pad,), N, jnp.int32)]).reshape(G, GROUP)

    degp = _deg_kernel(dstg)
    deg = 1.0 + degp[0, :N, 0] + degp[1, :N, 0]
    dinv = lax.rsqrt(deg)
    dinvM = jnp.broadcast_to(dinv[:, None], (N, D))

    hs = _mm1(x, W1, dinvM)
    aggp = _scatter_kernel(hs, srcg, dstg)
    h1 = _ew1(aggp[0, :N], aggp[1, :N], hs, dinvM, b1.reshape(1, D))
    nbp = _scatter_kernel(h1, srcg, dstg)
    out = _mlp(nbp[0, :N], nbp[1, :N], h1,
               Wl, Wr, bS.reshape(1, D),
               Wa, ba.reshape(1, D_MID),
               Wb, bb.reshape(1, D),
               Wc, bc.reshape(1, D_OUT))
    return out
